# baseline (speedup 1.0000x reference)
import functools

import jax
import jax.numpy as jnp
from jax import lax
from jax.experimental import pallas as pl
from jax.experimental.pallas import tpu as pltpu
from jax.experimental.pallas import tpu_sc as plsc

B, S, HIDDEN, FFN, E, TOPK = 1, 2048, 1024, 1024, 8, 2
T = B * S
TMG = 256
P = T * TOPK + E * TMG
NT = P // TMG
NC, NS = 2, 16
NW = NC * NS
TPW = T // NW


def _router_body(x_ref, wr_ref, pos_ref, w_ref, te_ref):
    logits = lax.dot_general(wr_ref[...], x_ref[...], (((1,), (1,)), ((), ())),
                             preferred_element_type=jnp.float32)
    eidx = lax.broadcasted_iota(jnp.int32, logits.shape, 0)
    m1 = jnp.max(logits, axis=0, keepdims=True)
    i1 = jnp.min(jnp.where(logits == m1, eidx, E), axis=0, keepdims=True)
    masked = jnp.where(eidx == i1, -jnp.inf, logits)
    m2 = jnp.max(masked, axis=0, keepdims=True)
    i2 = jnp.min(jnp.where(masked == m2, eidx, E), axis=0, keepdims=True)
    e2 = jnp.exp(m2 - m1)
    w1 = 1.0 / (1.0 + e2)
    w2 = e2 / (1.0 + e2)

    oh = (eidx == i1).astype(jnp.int32) + (eidx == i2).astype(jnp.int32)
    csum = oh
    d = 1
    while d < T:
        csum = csum + jnp.concatenate(
            [jnp.zeros((E, d), jnp.int32), csum[:, : T - d]], axis=1)
        d *= 2
    csum_excl = csum - oh
    cnt = csum[:, T - 1:T]
    padded = ((cnt + (TMG - 1)) // TMG) * TMG
    bacc = padded
    d = 1
    while d < E:
        bacc = bacc + jnp.concatenate(
            [jnp.zeros((d, 1), jnp.int32), bacc[: E - d]], axis=0)
        d *= 2
    base_excl = bacc - padded

    r0 = jnp.sum(jnp.where(eidx == i1, csum_excl, 0), axis=0)
    r1 = jnp.sum(jnp.where(eidx == i2, csum_excl, 0), axis=0)
    b0 = jnp.sum(jnp.where(eidx == i1, base_excl, 0), axis=0)
    b1 = jnp.sum(jnp.where(eidx == i2, base_excl, 0), axis=0)
    pos_ref[0, :] = b0 + r0
    pos_ref[1, :] = b1 + r1
    w_ref[0, :] = w1[0]
    w_ref[1, :] = w2[0]

    ends = jnp.broadcast_to(base_excl + padded, (E, NT))
    tstart = lax.broadcasted_iota(jnp.int32, (E, NT), 1) * TMG
    te = jnp.sum((ends <= tstart).astype(jnp.int32), axis=0)
    te_ref[0, :] = jnp.minimum(te, E - 1)


def _gmm_body(te_ref, xs_ref, w1_ref, w3_ref, w2_ref, out_ref):
    xb = xs_ref[...]
    g = lax.dot_general(xb, w1_ref[0], (((1,), (1,)), ((), ())),
                        preferred_element_type=jnp.float32,
                        precision=lax.Precision.DEFAULT)
    u = lax.dot_general(xb, w3_ref[0], (((1,), (1,)), ((), ())),
                        preferred_element_type=jnp.float32,
                        precision=lax.Precision.DEFAULT)
    h = g * lax.logistic(g) * u
    out_ref[...] = lax.dot_general(h, w2_ref[0], (((1,), (1,)), ((), ())),
                                   preferred_element_type=jnp.float32,
                                   precision=lax.Precision.DEFAULT)


@functools.lru_cache(maxsize=1)
def _sc_kernels():
    mesh = plsc.VectorSubcoreMesh(core_axis_name="c", subcore_axis_name="s")

    @functools.partial(
        pl.kernel, mesh=mesh,
        out_type=jax.ShapeDtypeStruct((P, HIDDEN), jnp.float32),
        scratch_types=[
            pltpu.VMEM((TOPK, TPW), jnp.int32),
            pltpu.VMEM((TPW, HIDDEN), jnp.float32),
            pltpu.SemaphoreType.DMA,
            pltpu.SemaphoreType.DMA,
        ],
    )
    def sc_dispatch(x_hbm, pos_hbm, xs_hbm, idx_v, rows_v, sem0, sem1):
        wid = lax.axis_index("s") * NC + lax.axis_index("c")
        base = wid * TPW
        pltpu.sync_copy(pos_hbm.at[wid], idx_v)
        pltpu.sync_copy(x_hbm.at[pl.ds(base, TPW)], rows_v)
        c0 = pltpu.async_copy(rows_v, xs_hbm.at[idx_v.at[0]], sem0)
        c1 = pltpu.async_copy(rows_v, xs_hbm.at[idx_v.at[1]], sem1)
        c0.wait()
        c1.wait()

    CH = 16

    @functools.partial(
        pl.kernel, mesh=mesh,
        out_type=jax.ShapeDtypeStruct((T, HIDDEN), jnp.float32),
        scratch_types=[
            pltpu.VMEM((TOPK, TPW), jnp.int32),
            pltpu.VMEM((TOPK, TPW), jnp.float32),
            pltpu.VMEM((CH, HIDDEN), jnp.float32),
            pltpu.VMEM((CH, HIDDEN), jnp.float32),
            pltpu.SemaphoreType.DMA,
            pltpu.SemaphoreType.DMA,
        ],
    )
    def sc_return_combine(ys_hbm, pos_hbm, w_hbm, o_hbm,
                          idx_v, w_v, a_v, b_v, sem0, sem1):
        wid = lax.axis_index("s") * NC + lax.axis_index("c")
        base = wid * TPW
        pltpu.sync_copy(pos_hbm.at[wid], idx_v)
        pltpu.sync_copy(w_hbm.at[wid], w_v)

        def _chunk(ch, _):
            sl = pl.ds(ch * CH, CH)
            c0 = pltpu.async_copy(ys_hbm.at[idx_v.at[0, sl]], a_v, sem0)
            c1 = pltpu.async_copy(ys_hbm.at[idx_v.at[1, sl]], b_v, sem1)
            c0.wait()
            c1.wait()
            wa_vec = w_v[0, pl.ds(ch * CH, CH)]
            wb_vec = w_v[1, pl.ds(ch * CH, CH)]
            for i in range(CH):
                wa = jnp.full((16,), wa_vec[i], jnp.float32)
                wb = jnp.full((16,), wb_vec[i], jnp.float32)
                for c in range(HIDDEN // 16):
                    cs = pl.ds(c * 16, 16)
                    a_v[i, cs] = a_v[i, cs] * wa + b_v[i, cs] * wb
            pltpu.sync_copy(a_v, o_hbm.at[pl.ds(base + ch * CH, CH)])
            return 0

        lax.fori_loop(0, TPW // CH, _chunk, 0)

    return sc_dispatch, sc_return_combine


@jax.jit
def kernel(x, Wr, W1, W3, W2):
    xf = x.reshape(T, HIDDEN)
    pos, w, te = pl.pallas_call(
        _router_body,
        out_shape=(
            jax.ShapeDtypeStruct((TOPK, T), jnp.int32),
            jax.ShapeDtypeStruct((TOPK, T), jnp.float32),
            jax.ShapeDtypeStruct((1, NT), jnp.int32),
        ),
    )(xf, Wr)

    sc_dispatch, sc_return_combine = _sc_kernels()
    pos3 = pos.reshape(TOPK, NW, TPW).transpose(1, 0, 2)
    xs = sc_dispatch(xf, pos3)

    ys = pl.pallas_call(
        _gmm_body,
        grid_spec=pltpu.PrefetchScalarGridSpec(
            num_scalar_prefetch=1,
            grid=(NT,),
            in_specs=[
                pl.BlockSpec((TMG, HIDDEN), lambda i, te_r: (i, 0)),
                pl.BlockSpec((1, FFN, HIDDEN), lambda i, te_r: (te_r[i], 0, 0)),
                pl.BlockSpec((1, FFN, HIDDEN), lambda i, te_r: (te_r[i], 0, 0)),
                pl.BlockSpec((1, HIDDEN, FFN), lambda i, te_r: (te_r[i], 0, 0)),
            ],
            out_specs=pl.BlockSpec((TMG, HIDDEN), lambda i, te_r: (i, 0)),
        ),
        out_shape=jax.ShapeDtypeStruct((P, HIDDEN), jnp.float32),
        compiler_params=pltpu.CompilerParams(
            dimension_semantics=("arbitrary",),
        ),
    )(te.reshape(NT), xs, W1, W3, W2)

    w3d = w.reshape(TOPK, NW, TPW).transpose(1, 0, 2)
    out = sc_return_combine(ys, pos3, w3d)
    return out.reshape(B, S, HIDDEN)


# speedup vs baseline: 1.1031x; 1.1031x over previous
import functools

import jax
import jax.numpy as jnp
from jax import lax
from jax.experimental import pallas as pl
from jax.experimental.pallas import tpu as pltpu
from jax.experimental.pallas import tpu_sc as plsc

B, S, HIDDEN, FFN, E, TOPK = 1, 2048, 1024, 1024, 8, 2
T = B * S
TMG = 256
P = T * TOPK + E * TMG
NT = P // TMG
NC, NS = 2, 16
NW = NC * NS
TPW = T // NW
TMC = 512


def _combine_body(w0_ref, w1_ref, a_ref, b_ref, o_ref):
    o_ref[...] = w0_ref[...] * a_ref[...] + w1_ref[...] * b_ref[...]


def _router_body(x_ref, wr_ref, pos_ref, w_ref, te_ref):
    logits = lax.dot_general(wr_ref[...], x_ref[...], (((1,), (1,)), ((), ())),
                             preferred_element_type=jnp.float32)
    eidx = lax.broadcasted_iota(jnp.int32, logits.shape, 0)
    m1 = jnp.max(logits, axis=0, keepdims=True)
    i1 = jnp.min(jnp.where(logits == m1, eidx, E), axis=0, keepdims=True)
    masked = jnp.where(eidx == i1, -jnp.inf, logits)
    m2 = jnp.max(masked, axis=0, keepdims=True)
    i2 = jnp.min(jnp.where(masked == m2, eidx, E), axis=0, keepdims=True)
    e2 = jnp.exp(m2 - m1)
    w1 = 1.0 / (1.0 + e2)
    w2 = e2 / (1.0 + e2)

    oh = (eidx == i1).astype(jnp.int32) + (eidx == i2).astype(jnp.int32)
    csum = oh
    d = 1
    while d < T:
        csum = csum + jnp.concatenate(
            [jnp.zeros((E, d), jnp.int32), csum[:, : T - d]], axis=1)
        d *= 2
    csum_excl = csum - oh
    cnt = csum[:, T - 1:T]
    padded = ((cnt + (TMG - 1)) // TMG) * TMG
    bacc = padded
    d = 1
    while d < E:
        bacc = bacc + jnp.concatenate(
            [jnp.zeros((d, 1), jnp.int32), bacc[: E - d]], axis=0)
        d *= 2
    base_excl = bacc - padded

    r0 = jnp.sum(jnp.where(eidx == i1, csum_excl, 0), axis=0)
    r1 = jnp.sum(jnp.where(eidx == i2, csum_excl, 0), axis=0)
    b0 = jnp.sum(jnp.where(eidx == i1, base_excl, 0), axis=0)
    b1 = jnp.sum(jnp.where(eidx == i2, base_excl, 0), axis=0)
    pos_ref[0, :] = b0 + r0
    pos_ref[1, :] = b1 + r1
    w_ref[0, :] = w1[0]
    w_ref[1, :] = w2[0]

    ends = jnp.broadcast_to(base_excl + padded, (E, NT))
    tstart = lax.broadcasted_iota(jnp.int32, (E, NT), 1) * TMG
    te = jnp.sum((ends <= tstart).astype(jnp.int32), axis=0)
    te_ref[0, :] = jnp.minimum(te, E - 1)


def _gmm_body(te_ref, xs_ref, w1_ref, w3_ref, w2_ref, out_ref):
    xb = xs_ref[...]
    g = lax.dot_general(xb, w1_ref[0], (((1,), (1,)), ((), ())),
                        preferred_element_type=jnp.float32,
                        precision=lax.Precision.DEFAULT)
    u = lax.dot_general(xb, w3_ref[0], (((1,), (1,)), ((), ())),
                        preferred_element_type=jnp.float32,
                        precision=lax.Precision.DEFAULT)
    h = g * lax.logistic(g) * u
    out_ref[...] = lax.dot_general(h, w2_ref[0], (((1,), (1,)), ((), ())),
                                   preferred_element_type=jnp.float32,
                                   precision=lax.Precision.DEFAULT)


@functools.lru_cache(maxsize=1)
def _sc_kernels():
    mesh = plsc.VectorSubcoreMesh(core_axis_name="c", subcore_axis_name="s")

    @functools.partial(
        pl.kernel, mesh=mesh,
        out_type=jax.ShapeDtypeStruct((P, HIDDEN), jnp.float32),
        scratch_types=[
            pltpu.VMEM((TOPK, TPW), jnp.int32),
            pltpu.VMEM((TPW, HIDDEN), jnp.float32),
            pltpu.SemaphoreType.DMA,
            pltpu.SemaphoreType.DMA,
        ],
    )
    def sc_dispatch(x_hbm, pos_hbm, xs_hbm, idx_v, rows_v, sem0, sem1):
        wid = lax.axis_index("s") * NC + lax.axis_index("c")
        base = wid * TPW
        pltpu.sync_copy(pos_hbm.at[wid], idx_v)
        pltpu.sync_copy(x_hbm.at[pl.ds(base, TPW)], rows_v)
        c0 = pltpu.async_copy(rows_v, xs_hbm.at[idx_v.at[0]], sem0)
        c1 = pltpu.async_copy(rows_v, xs_hbm.at[idx_v.at[1]], sem1)
        c0.wait()
        c1.wait()

    @functools.partial(
        pl.kernel, mesh=mesh,
        out_type=(jax.ShapeDtypeStruct((T, HIDDEN), jnp.float32),
                  jax.ShapeDtypeStruct((T, HIDDEN), jnp.float32)),
        scratch_types=[
            pltpu.VMEM((TOPK, TPW), jnp.int32),
            pltpu.VMEM((TPW, HIDDEN), jnp.float32),
            pltpu.SemaphoreType.DMA,
        ],
    )
    def sc_return_gather(ys_hbm, pos_hbm, a_hbm, b_hbm, idx_v, rows_v, sem):
        wid = lax.axis_index("s") * NC + lax.axis_index("c")
        base = wid * TPW
        pltpu.sync_copy(pos_hbm.at[wid], idx_v)
        pltpu.async_copy(ys_hbm.at[idx_v.at[0]], rows_v, sem).wait()
        pltpu.sync_copy(rows_v, a_hbm.at[pl.ds(base, TPW)])
        pltpu.async_copy(ys_hbm.at[idx_v.at[1]], rows_v, sem).wait()
        pltpu.sync_copy(rows_v, b_hbm.at[pl.ds(base, TPW)])

    return sc_dispatch, sc_return_gather


@jax.jit
def kernel(x, Wr, W1, W3, W2):
    xf = x.reshape(T, HIDDEN)
    pos, w, te = pl.pallas_call(
        _router_body,
        out_shape=(
            jax.ShapeDtypeStruct((TOPK, T), jnp.int32),
            jax.ShapeDtypeStruct((TOPK, T), jnp.float32),
            jax.ShapeDtypeStruct((1, NT), jnp.int32),
        ),
    )(xf, Wr)

    sc_dispatch, sc_return_gather = _sc_kernels()
    pos3 = pos.reshape(TOPK, NW, TPW).transpose(1, 0, 2)
    xs = sc_dispatch(xf, pos3)

    ys = pl.pallas_call(
        _gmm_body,
        grid_spec=pltpu.PrefetchScalarGridSpec(
            num_scalar_prefetch=1,
            grid=(NT,),
            in_specs=[
                pl.BlockSpec((TMG, HIDDEN), lambda i, te_r: (i, 0)),
                pl.BlockSpec((1, FFN, HIDDEN), lambda i, te_r: (te_r[i], 0, 0)),
                pl.BlockSpec((1, FFN, HIDDEN), lambda i, te_r: (te_r[i], 0, 0)),
                pl.BlockSpec((1, HIDDEN, FFN), lambda i, te_r: (te_r[i], 0, 0)),
            ],
            out_specs=pl.BlockSpec((TMG, HIDDEN), lambda i, te_r: (i, 0)),
        ),
        out_shape=jax.ShapeDtypeStruct((P, HIDDEN), jnp.float32),
        compiler_params=pltpu.CompilerParams(
            dimension_semantics=("arbitrary",),
        ),
    )(te.reshape(NT), xs, W1, W3, W2)

    a, b = sc_return_gather(ys, pos3)

    out = pl.pallas_call(
        _combine_body,
        grid=(T // TMC,),
        in_specs=[
            pl.BlockSpec((TMC, 1), lambda i: (i, 0)),
            pl.BlockSpec((TMC, 1), lambda i: (i, 0)),
            pl.BlockSpec((TMC, HIDDEN), lambda i: (i, 0)),
            pl.BlockSpec((TMC, HIDDEN), lambda i: (i, 0)),
        ],
        out_specs=pl.BlockSpec((TMC, HIDDEN), lambda i: (i, 0)),
        out_shape=jax.ShapeDtypeStruct((T, HIDDEN), jnp.float32),
    )(w[0].reshape(T, 1), w[1].reshape(T, 1), a, b)
    return out.reshape(B, S, HIDDEN)
